# baseline (speedup 1.0000x reference)
import jax
import jax.numpy as jnp
from jax.experimental import pallas as pl
from jax.experimental.pallas import tpu as pltpu

K = 2048
N = 16
BJ = 2048
NJ = K // BJ
RCHUNK = 256
NCHUNK = K // RCHUNK
CPS = NCHUNK // NJ


def _rank_chunk(col, xT_ref, rank_ref, m, bank, c):
    lo, hi = c * RCHUNK, (c + 1) * RCHUNK
    xTc = xT_ref[lo:hi, :]
    lane = jax.lax.broadcasted_iota(jnp.int32, (RCHUNK, N), 1)
    xc = jnp.sum(jnp.where(lane == m, xTc, 0.0), axis=1,
                 keepdims=True)
    parts = []
    if lo > 0:
        parts.append((xc > col[:, :lo]).astype(jnp.float32))
    cold = col[:, lo:hi]
    tri = jax.lax.broadcasted_iota(jnp.int32, (RCHUNK, RCHUNK), 0) < \
        jax.lax.broadcasted_iota(jnp.int32, (RCHUNK, RCHUNK), 1)
    td = (xc > cold) | ((xc == cold) & tri)
    parts.append(td.astype(jnp.float32))
    if hi < K:
        parts.append((xc >= col[:, hi:]).astype(jnp.float32))
    maskf = jnp.concatenate(parts, axis=1)
    ones8 = jnp.ones((8, RCHUNK), jnp.float32)
    part8 = jax.lax.dot_general(
        ones8, maskf, (((1,), (0,)), ((), ())),
        preferred_element_type=jnp.float32)
    part = part8[0:1]

    if c == 0:
        rank_ref[bank] = part
    else:
        rank_ref[bank] = rank_ref[bank] + part


def _perm_kernel(x_ref, xT_ref, out_ref, rank_ref, rows_ref):
    n = pl.program_id(0)
    j = pl.program_id(1)
    parity = jax.lax.rem(n, 2)

    @pl.when((n == 0) & (j == 0))
    def _prologue():
        rows_ref[...] = jax.lax.broadcasted_iota(
            jnp.int32, (BJ, K), 0).astype(jnp.float32)
        col = x_ref[0]
        for c in range(NCHUNK):
            _rank_chunk(col, xT_ref, rank_ref, 0, 0, c)

    rank_s = rank_ref[parity] - (j * BJ).astype(jnp.float32)
    out_ref[0] = (rows_ref[...] == rank_s).astype(jnp.float32)

    @pl.when(n < N - 1)
    def _next_row():
        col = x_ref[n + 1]
        for jj in range(NJ):
            @pl.when(j == jj)
            def _chunks(jj=jj, col=col):
                for t in range(CPS):
                    _rank_chunk(col, xT_ref, rank_ref, n + 1, 1 - parity,
                                jj * CPS + t)


@jax.jit
def kernel(input):
    n_, k_ = input.shape
    assert (n_, k_) == (N, K)
    x3 = input.reshape(N, 1, K)
    xT = input.T
    return pl.pallas_call(
        _perm_kernel,
        grid=(N, NJ),
        in_specs=[
            pl.BlockSpec((N, 1, K), lambda n, j: (0, 0, 0)),
            pl.BlockSpec((K, N), lambda n, j: (0, 0)),
        ],
        out_specs=pl.BlockSpec((1, BJ, K), lambda n, j: (n, j, 0)),
        out_shape=jax.ShapeDtypeStruct((N, K, K), input.dtype),
        scratch_shapes=[pltpu.VMEM((2, 1, K), jnp.float32),
                        pltpu.VMEM((BJ, K), jnp.float32)],
        compiler_params=pltpu.CompilerParams(
            dimension_semantics=("arbitrary", "arbitrary"),
        ),
    )(x3, xT)


# speedup vs baseline: 1.0335x; 1.0335x over previous
import jax
import jax.numpy as jnp
from jax.experimental import pallas as pl
from jax.experimental.pallas import tpu as pltpu

K = 2048
N = 16
BJ = 1024
NJ = K // BJ
RCHUNK = 256
NCHUNK = K // RCHUNK
CPS = NCHUNK // NJ


def _rank_chunk(col, xT_ref, rank_ref, m, bank, c):
    lo, hi = c * RCHUNK, (c + 1) * RCHUNK
    xTc = xT_ref[lo:hi, :]
    lane = jax.lax.broadcasted_iota(jnp.int32, (RCHUNK, N), 1)
    xc = jnp.sum(jnp.where(lane == m, xTc, 0.0), axis=1,
                 keepdims=True)
    parts = []
    if lo > 0:
        parts.append((xc > col[:, :lo]).astype(jnp.float32))
    cold = col[:, lo:hi]
    tri = jax.lax.broadcasted_iota(jnp.int32, (RCHUNK, RCHUNK), 0) < \
        jax.lax.broadcasted_iota(jnp.int32, (RCHUNK, RCHUNK), 1)
    td = (xc > cold) | ((xc == cold) & tri)
    parts.append(td.astype(jnp.float32))
    if hi < K:
        parts.append((xc >= col[:, hi:]).astype(jnp.float32))
    maskf = jnp.concatenate(parts, axis=1)
    ones8 = jnp.ones((8, RCHUNK), jnp.float32)
    part8 = jax.lax.dot_general(
        ones8, maskf, (((1,), (0,)), ((), ())),
        preferred_element_type=jnp.float32)
    part = part8[0:1]

    if c == 0:
        rank_ref[bank] = part
    else:
        rank_ref[bank] = rank_ref[bank] + part


def _perm_kernel(x_ref, xT_ref, out_ref, rank_ref, rows_ref):
    n = pl.program_id(0)
    j = pl.program_id(1)
    parity = jax.lax.rem(n, 2)

    @pl.when((n == 0) & (j == 0))
    def _prologue():
        rows_ref[...] = jax.lax.broadcasted_iota(
            jnp.int32, (BJ, K), 0).astype(jnp.float32)
        col = x_ref[0]
        for c in range(NCHUNK):
            _rank_chunk(col, xT_ref, rank_ref, 0, 0, c)

    rank_s = rank_ref[parity] - (j * BJ).astype(jnp.float32)
    out_ref[0] = (rows_ref[...] == rank_s).astype(jnp.float32)

    @pl.when(n < N - 1)
    def _next_row():
        col = x_ref[n + 1]
        for jj in range(NJ):
            @pl.when(j == jj)
            def _chunks(jj=jj, col=col):
                for t in range(CPS):
                    _rank_chunk(col, xT_ref, rank_ref, n + 1, 1 - parity,
                                jj * CPS + t)


@jax.jit
def kernel(input):
    n_, k_ = input.shape
    assert (n_, k_) == (N, K)
    x3 = input.reshape(N, 1, K)
    xT = input.T
    return pl.pallas_call(
        _perm_kernel,
        grid=(N, NJ),
        in_specs=[
            pl.BlockSpec((N, 1, K), lambda n, j: (0, 0, 0)),
            pl.BlockSpec((K, N), lambda n, j: (0, 0)),
        ],
        out_specs=pl.BlockSpec((1, BJ, K), lambda n, j: (n, j, 0)),
        out_shape=jax.ShapeDtypeStruct((N, K, K), input.dtype),
        scratch_shapes=[pltpu.VMEM((2, 1, K), jnp.float32),
                        pltpu.VMEM((BJ, K), jnp.float32)],
        compiler_params=pltpu.CompilerParams(
            dimension_semantics=("arbitrary", "arbitrary"),
        ),
    )(x3, xT)


# speedup vs baseline: 1.0379x; 1.0043x over previous
import jax
import jax.numpy as jnp
from jax.experimental import pallas as pl
from jax.experimental.pallas import tpu as pltpu

K = 2048
N = 16
BJ = 1024
NJ = K // BJ
RCHUNK = 256
NCHUNK = K // RCHUNK
CPS = NCHUNK // NJ


def _rank_chunk(col, xT_ref, rank_ref, rank_i_ref, m, bank, c):
    lo, hi = c * RCHUNK, (c + 1) * RCHUNK
    xTc = xT_ref[lo:hi, :]
    lane = jax.lax.broadcasted_iota(jnp.int32, (RCHUNK, N), 1)
    xc = jnp.sum(jnp.where(lane == m, xTc, 0.0), axis=1,
                 keepdims=True)
    parts = []
    if lo > 0:
        parts.append((xc > col[:, :lo]).astype(jnp.float32))
    cold = col[:, lo:hi]
    tri = jax.lax.broadcasted_iota(jnp.int32, (RCHUNK, RCHUNK), 0) < \
        jax.lax.broadcasted_iota(jnp.int32, (RCHUNK, RCHUNK), 1)
    td = (xc > cold) | ((xc == cold) & tri)
    parts.append(td.astype(jnp.float32))
    if hi < K:
        parts.append((xc >= col[:, hi:]).astype(jnp.float32))
    maskf = jnp.concatenate(parts, axis=1)
    ones8 = jnp.ones((8, RCHUNK), jnp.float32)
    part8 = jax.lax.dot_general(
        ones8, maskf, (((1,), (0,)), ((), ())),
        preferred_element_type=jnp.float32)
    part = part8[0:1]

    if c == 0:
        rank_ref[bank] = part
    elif c == NCHUNK - 1:
        rank_i_ref[bank] = (rank_ref[bank] + part).astype(jnp.int32)
    else:
        rank_ref[bank] = rank_ref[bank] + part


def _perm_kernel(x_ref, xT_ref, out_ref, rank_ref, rank_i_ref):
    n = pl.program_id(0)
    j = pl.program_id(1)
    parity = jax.lax.rem(n, 2)

    @pl.when((n == 0) & (j == 0))
    def _prologue():
        col = x_ref[0]
        for c in range(NCHUNK):
            _rank_chunk(col, xT_ref, rank_ref, rank_i_ref, 0, 0, c)

    rank_s = rank_i_ref[parity] - j * BJ
    row = jax.lax.broadcasted_iota(jnp.int32, (BJ, K), 0)
    out_ref[0] = (row == rank_s).astype(jnp.float32)

    @pl.when(n < N - 1)
    def _next_row():
        col = x_ref[n + 1]
        for jj in range(NJ):
            @pl.when(j == jj)
            def _chunks(jj=jj, col=col):
                for t in range(CPS):
                    _rank_chunk(col, xT_ref, rank_ref, rank_i_ref,
                                n + 1, 1 - parity, jj * CPS + t)


@jax.jit
def kernel(input):
    n_, k_ = input.shape
    assert (n_, k_) == (N, K)
    x3 = input.reshape(N, 1, K)
    xT = input.T
    return pl.pallas_call(
        _perm_kernel,
        grid=(N, NJ),
        in_specs=[
            pl.BlockSpec((N, 1, K), lambda n, j: (0, 0, 0)),
            pl.BlockSpec((K, N), lambda n, j: (0, 0)),
        ],
        out_specs=pl.BlockSpec((1, BJ, K), lambda n, j: (n, j, 0)),
        out_shape=jax.ShapeDtypeStruct((N, K, K), input.dtype),
        scratch_shapes=[pltpu.VMEM((2, 1, K), jnp.float32),
                        pltpu.VMEM((2, 1, K), jnp.int32)],
        compiler_params=pltpu.CompilerParams(
            dimension_semantics=("arbitrary", "arbitrary"),
        ),
    )(x3, xT)
